# baseline (speedup 1.0000x reference)
import jax
import jax.numpy as jnp
from jax import lax
from jax.experimental import pallas as pl
from jax.experimental.pallas import tpu as pltpu

RADIUS, DECAY_RADIUS = 0.15, 0.3
EPS_BN = 1e-5
B, N = 4, 1024
C_IN, C_OUT, C_BYP = 64, 64, 32
C6 = 6 * C_BYP
TI = 256
BN_CNT = B * N
_R2 = RADIUS * RADIUS
_DR2 = DECAY_RADIUS * DECAY_RADIUS
_C1 = 1.0 / (_DR2 - _R2)
_C0 = _DR2 * _C1


def _bn_fold(x, gamma, beta):
    s1 = jnp.sum(x, axis=0, keepdims=True)
    s2 = jnp.sum(x * x, axis=0, keepdims=True)
    mean = s1 * (1.0 / BN_CNT)
    var = s2 * (1.0 / BN_CNT) - mean * mean
    a = gamma * lax.rsqrt(var + EPS_BN)
    return a, beta - mean * a


def _k_all(feat_ref, xyz_ref, xyzt_ref, wf_ref, bf_ref, wb_ref, gb_ref, beb_ref,
           wag_ref, bag_ref, g1_ref, b1_ref, g2_ref, b2_ref, out_ref, ag_scr):
    feat = feat_ref[...]
    self_feat = jnp.dot(feat, wf_ref[...], preferred_element_type=jnp.float32) + bf_ref[...]
    mut = jnp.dot(feat, wb_ref[...], preferred_element_type=jnp.float32)

    a_b, sh_b = _bn_fold(mut, gb_ref[...], beb_ref[...])

    ones_t = jnp.ones((TI, C_BYP), dtype=jnp.float32)
    nt = N // TI
    for b in range(B):
        g = jnp.maximum(mut[b * N:(b + 1) * N, :] * a_b + sh_b, 0.0)
        gdiff = [(g[:, (2 * ax) * C_BYP:(2 * ax + 1) * C_BYP]
                  - g[:, (2 * ax + 1) * C_BYP:(2 * ax + 2) * C_BYP]
                  ).astype(jnp.bfloat16) for ax in range(3)]
        gm = [g[:, (2 * ax + 1) * C_BYP:(2 * ax + 2) * C_BYP].astype(jnp.bfloat16)
              for ax in range(3)]
        xj = xyzt_ref[b]

        accs = [jnp.zeros((TI, C_BYP), dtype=jnp.float32) for _ in range(nt)]
        norms = [jnp.zeros((TI, C_BYP), dtype=jnp.float32) for _ in range(nt)]
        for ii in range(nt):
            xi = xyz_ref[b, ii * TI:(ii + 1) * TI, :]
            for jj in range(ii, nt):
                xr = xj[:, jj * TI:(jj + 1) * TI]
                dx = xr[0:1, :] - xi[:, 0:1]
                dy = xr[1:2, :] - xi[:, 1:2]
                dz = xr[2:3, :] - xi[:, 2:3]
                sqx = dx * dx
                sqy = dy * dy
                sqz = dz * dz
                dist2 = sqx + sqy + sqz
                d2c = jnp.maximum(dist2, 1e-12)
                rcp = pl.reciprocal(d2c, approx=True)
                w = jnp.clip(_C0 - dist2 * _C1, 0.0, 1.0)
                u = w * rcp
                jsl = slice(jj * TI, (jj + 1) * TI)
                isl = slice(ii * TI, (ii + 1) * TI)
                for ax, (d, sq) in enumerate(((dx, sqx), (dy, sqy), (dz, sqz))):
                    q = u * sq
                    ap = jnp.where(d > 0.0, q, 0.0)
                    accs[ii] += jnp.dot(ap.astype(jnp.bfloat16), gdiff[ax][jsl],
                                        preferred_element_type=jnp.float32)
                    accs[ii] += jnp.dot(q.astype(jnp.bfloat16), gm[ax][jsl],
                                        preferred_element_type=jnp.float32)
                    if jj > ii:
                        qt = q.T
                        apt = qt - ap.T
                        accs[jj] += jnp.dot(apt.astype(jnp.bfloat16), gdiff[ax][isl],
                                            preferred_element_type=jnp.float32)
                        accs[jj] += jnp.dot(qt.astype(jnp.bfloat16), gm[ax][isl],
                                            preferred_element_type=jnp.float32)
                norms[ii] += jnp.dot(w, ones_t, preferred_element_type=jnp.float32)
                if jj > ii:
                    norms[jj] += jnp.dot(w.T, ones_t,
                                         preferred_element_type=jnp.float32)
        for ii in range(nt):
            ag_scr[b * N + ii * TI:b * N + (ii + 1) * TI, :] = (
                accs[ii] / jnp.maximum(norms[ii] - 1.0, 1e-8))

    ag = ag_scr[...]
    a1, sh1 = _bn_fold(ag, g1_ref[...], b1_ref[...])
    agn = jnp.maximum(ag * a1 + sh1, 0.0)
    pre = (jnp.dot(agn, wag_ref[...], preferred_element_type=jnp.float32)
           + bag_ref[...] + self_feat)
    a2, sh2 = _bn_fold(pre, g2_ref[...], b2_ref[...])
    out_ref[...] = jnp.maximum(pre * a2 + sh2, 0.0)


def kernel(feat, xyz, W_feat, b_feat, W_byp, g_byp, be_byp, W_ag, b_ag, g1, b1, g2, b2):
    out = pl.pallas_call(
        _k_all,
        out_shape=jax.ShapeDtypeStruct((B * N, C_OUT), jnp.float32),
        scratch_shapes=[pltpu.VMEM((B * N, C_BYP), jnp.float32)],
    )(feat.reshape(B * N, C_IN), xyz, jnp.transpose(xyz, (0, 2, 1)),
      W_feat.T, b_feat.reshape(1, C_OUT), W_byp.T,
      g_byp.reshape(1, C6), be_byp.reshape(1, C6),
      W_ag.T, b_ag.reshape(1, C_OUT),
      g1.reshape(1, C_BYP), b1.reshape(1, C_BYP),
      g2.reshape(1, C_OUT), b2.reshape(1, C_OUT))
    return out.reshape(B, N, C_OUT)
